# Initial kernel scaffold; baseline (speedup 1.0000x reference)
import jax
import jax.numpy as jnp
from jax.experimental import pallas as pl


def kernel(x, edge_index, W1, att_src1, att_dst1, b1, W2, att_src2, att_dst2, b2):
    raise NotImplementedError("write your pallas kernel here")


# speedup vs baseline: 10.9055x; 10.9055x over previous
import functools

import jax
import jax.numpy as jnp
from jax import lax
from jax.experimental import pallas as pl
from jax.experimental.pallas import tpu as pltpu
from jax.experimental.pallas import tpu_sc as plsc

N = 10000
NPAD = 10240
NFEAT = 128
HEADS = 8
NHID = 64
H1 = HEADS * NHID
NCLASS = 40
C2 = 48
BN = 1024
BNE = 4096
F32 = jnp.float32
PREC = lax.Precision.HIGHEST


def _dot3(a, b):
    bf = jnp.bfloat16
    return jnp.dot(a.astype(bf), b.astype(bf), preferred_element_type=F32)


def _tc1_body(x_ref, w_ref, afs_ref, afd_ref, h_ref, as_ref, ad_ref):
    h = _dot3(x_ref[...], w_ref[...])
    as_ref[...] = _dot3(h, afs_ref[...])
    ad_ref[...] = _dot3(h, afd_ref[...])
    for ci in range(8):
        h_ref[ci] = h[:, ci * 64:(ci + 1) * 64]


def _tcx_body(e_ref, x_ref):
    e = e_ref[...]
    e = jnp.where(e >= 0.0, e, 0.2 * e)
    x_ref[...] = jnp.exp(e)


def _tc2_body(h_ref, b1_ref, w2_ref, afs_ref, afd_ref,
              g_ref, as_ref, ad_ref):
    hc = h_ref[...] + b1_ref[...]
    hc = jnp.where(hc > 0.0, hc, jnp.exp(hc) - 1.0)
    acc = _dot3(hc, w2_ref[...])
    g_ref[...] = acc
    as_ref[...] = _dot3(acc, afs_ref[...])
    ad_ref[...] = _dot3(acc, afd_ref[...])


def _tc3_body(pa_ref, pb_ref, b2_ref, o_ref):
    z = pa_ref[...] + pb_ref[...]
    z = z[:, :NCLASS] + b2_ref[...]
    m = jnp.max(z, axis=1, keepdims=True)
    o_ref[...] = z - m - jnp.log(jnp.sum(jnp.exp(z - m), axis=1,
                                         keepdims=True))


_GDN = lax.GatherDimensionNumbers(offset_dims=(), collapsed_slice_dims=(0,),
                                  start_index_map=(0,))


def _splat(vec, lane):
    idx = jnp.full((16, 1), lane, dtype=jnp.int32)
    return lax.gather(vec, idx, _GDN, (1,),
                      mode=lax.GatherScatterMode.PROMISE_IN_BOUNDS)


def _load_rows(hbm, vbuf, ria, rib, base, nrows):
    iota = lax.iota(jnp.int32, 16)
    n1 = min(nrows, 128)
    for v in range((n1 + 15) // 16):
        ria[pl.ds(v * 16, 16)] = iota + (base + v * 16)
    pltpu.sync_copy(hbm.at[ria.at[pl.ds(0, n1)]], vbuf.at[pl.ds(0, n1)])
    if nrows > 128:
        n2 = nrows - 128
        for v in range((n2 + 15) // 16):
            rib[pl.ds(v * 16, 16)] = iota + (base + 128 + v * 16)
        pltpu.sync_copy(hbm.at[rib.at[pl.ds(0, n2)]],
                        vbuf.at[pl.ds(128, n2)])


def _recip(d):
    r0 = 1.0 / d
    return r0 * (2.0 - d * r0)


def _sce_body(RW, asrc_hbm, adst_hbm, src_hbm, dst_hbm, e_hbm,
              srcw, dstw, arow, brow, erow, ria, rib):
    c = lax.axis_index("c")
    t = lax.axis_index("s")
    w = c * 16 + t
    _load_rows(src_hbm, srcw, ria, rib, w * RW, RW)
    _load_rows(dst_hbm, dstw, ria, rib, w * RW, RW)

    def _pe(b, carry):
        pltpu.sync_copy(asrc_hbm.at[srcw.at[b]], arow)
        pltpu.sync_copy(adst_hbm.at[dstw.at[b]], brow)

        def _cmp(i, cc):
            erow[i] = arow[i] + brow[i]
            return cc
        lax.fori_loop(0, 128, _cmp, 0)
        pltpu.sync_copy(erow, e_hbm.at[pl.ds((w * RW + b) * 128, 128)])
        return carry
    lax.fori_loop(0, RW, _pe, 0)


def _scm1_body(RT, hflat_hbm, ex_hbm, src_hbm, dst_hbm, z16_hbm, z64_hbm,
               out_hbm, denom_sp, acc_sp, srcv, dstv, exv, dnv, rowsv,
               gidxv, ria, rib):
    c = lax.axis_index("c")
    t = lax.axis_index("s")

    def _zd(j, carry):
        pltpu.sync_copy(z16_hbm, denom_sp.at[pl.ds(t * 640 + j * 128, 128)])
        return carry
    lax.fori_loop(0, 5, _zd, 0)

    _load_rows(src_hbm, srcv, ria, rib, t * RT, RT)
    _load_rows(dst_hbm, dstv, ria, rib, t * RT, RT)
    plsc.subcore_barrier()

    def _pa(b, carry):
        pltpu.sync_copy(ex_hbm.at[pl.ds((t * RT + b) * 128, 128)], exv)
        pltpu.sync_copy(exv, denom_sp.at[dstv.at[b]], add=True)
        return carry
    lax.fori_loop(0, RT, _pa, 0)
    plsc.subcore_barrier()

    for k in range(4):
        g = 4 * c + k

        def _za(j, carry):
            pltpu.sync_copy(z64_hbm, acc_sp.at[pl.ds(t * 640 + j * 128, 128)])
            return carry
        lax.fori_loop(0, 5, _za, 0)
        plsc.subcore_barrier()

        def _pb(b, carry):
            def _gi(i, cc):
                gidxv[pl.ds(i * 16, 16)] = (
                    srcv[b, pl.ds(i * 16, 16)] + g * NPAD)
                return cc
            lax.fori_loop(0, 8, _gi, 0)
            pltpu.sync_copy(hflat_hbm.at[gidxv], rowsv)
            pltpu.sync_copy(ex_hbm.at[pl.ds((t * RT + b) * 128, 128)], exv)
            pltpu.sync_copy(denom_sp.at[dstv.at[b]], dnv)

            def _sc(i, cc):
                al = exv[i] * _recip(dnv[i] + 1e-16)
                s0 = _splat(al, g)
                for v in range(4):
                    rowsv[i, pl.ds(v * 16, 16)] = (
                        rowsv[i, pl.ds(v * 16, 16)] * s0)
                return cc
            lax.fori_loop(0, 128, _sc, 0)
            pltpu.sync_copy(rowsv, acc_sp.at[dstv.at[b]], add=True)
            return carry
        lax.fori_loop(0, RT, _pb, 0)
        plsc.subcore_barrier()

        def _wr(j, carry):
            r0 = t * 640 + j * 128
            pltpu.sync_copy(acc_sp.at[pl.ds(r0, 128)],
                            out_hbm.at[g, pl.ds(r0, 128)])
            return carry
        lax.fori_loop(0, 5, _wr, 0)
        if k < 3:
            plsc.subcore_barrier()


def _scm2_body(RT, RW, g_hbm, ex_hbm, src_hbm, dst_hbm, z16_hbm, z48_hbm,
               out_hbm, denom_sp, acc_sp, dstv, srcw, dstw, exv, dnv,
               rowsv, ria, rib):
    c = lax.axis_index("c")
    t = lax.axis_index("s")
    w = c * 16 + t

    def _zd(j, carry):
        pltpu.sync_copy(z16_hbm, denom_sp.at[pl.ds(t * 640 + j * 128, 128)])
        return carry
    lax.fori_loop(0, 5, _zd, 0)

    def _za(j, carry):
        pltpu.sync_copy(z48_hbm, acc_sp.at[pl.ds(t * 640 + j * 128, 128)])
        return carry
    lax.fori_loop(0, 5, _za, 0)

    _load_rows(dst_hbm, dstv, ria, rib, t * RT, RT)
    _load_rows(src_hbm, srcw, ria, rib, w * RW, RW)
    _load_rows(dst_hbm, dstw, ria, rib, w * RW, RW)
    plsc.subcore_barrier()

    def _pa(b, carry):
        pltpu.sync_copy(ex_hbm.at[pl.ds((t * RT + b) * 128, 128)], exv)
        pltpu.sync_copy(exv, denom_sp.at[dstv.at[b]], add=True)
        return carry
    lax.fori_loop(0, RT, _pa, 0)
    plsc.subcore_barrier()

    def _pb(b, carry):
        pltpu.sync_copy(g_hbm.at[srcw.at[b]], rowsv)
        pltpu.sync_copy(ex_hbm.at[pl.ds((w * RW + b) * 128, 128)], exv)
        pltpu.sync_copy(denom_sp.at[dstw.at[b]], dnv)

        def _sc(i, cc):
            al = exv[i] * _recip(dnv[i] + 1e-16)
            s0 = _splat(al, 0)
            for v in range(3):
                rowsv[i, pl.ds(v * 16, 16)] = (
                    rowsv[i, pl.ds(v * 16, 16)] * s0)
            return cc
        lax.fori_loop(0, 128, _sc, 0)
        pltpu.sync_copy(rowsv, acc_sp.at[dstw.at[b]], add=True)
        return carry
    lax.fori_loop(0, RW, _pb, 0)
    plsc.subcore_barrier()

    def _wr(j, carry):
        r0 = t * 640 + j * 128
        pltpu.sync_copy(acc_sp.at[pl.ds(r0, 128)],
                        out_hbm.at[c, pl.ds(r0, 128)])
        return carry
    lax.fori_loop(0, 5, _wr, 0)


def kernel(x, edge_index, W1, att_src1, att_dst1, b1, W2, att_src2,
           att_dst2, b2):
    n = x.shape[0]
    x_pad = jnp.pad(x, ((0, NPAD - n), (0, 0)))

    loop = jnp.arange(n, dtype=edge_index.dtype)
    ei = jnp.concatenate([edge_index, jnp.stack([loop, loop], axis=0)],
                         axis=1)
    e_tot = ei.shape[1]
    epad = ((e_tot + 4095) // 4096) * 4096
    fill = jnp.full((2, epad - e_tot), NPAD - 1, dtype=jnp.int32)
    ei = jnp.concatenate([ei.astype(jnp.int32), fill], axis=1)
    RT = epad // (16 * 128)
    RW = epad // (32 * 128)
    srcf = ei[0].reshape(epad // 128, 128)
    dstf = ei[1].reshape(epad // 128, 128)

    mesh = plsc.VectorSubcoreMesh(core_axis_name="c", subcore_axis_name="s")
    cp = pltpu.CompilerParams(use_tc_tiling_on_sc=False)
    z16 = jnp.zeros((128, 16), F32)
    z48 = jnp.zeros((128, C2), F32)
    z64 = jnp.zeros((128, 64), F32)

    eye8 = jnp.eye(8, 16, dtype=F32)
    afs1 = (att_src1[0][:, :, None] * eye8[:, None, :]).reshape(H1, 16)
    afd1 = (att_dst1[0][:, :, None] * eye8[:, None, :]).reshape(H1, 16)

    grid = NPAD // BN
    h1c, as1, ad1 = pl.pallas_call(
        _tc1_body,
        grid=(grid,),
        in_specs=[
            pl.BlockSpec((BN, NFEAT), lambda i: (i, 0)),
            pl.BlockSpec((NFEAT, H1), lambda i: (0, 0)),
            pl.BlockSpec((H1, 16), lambda i: (0, 0)),
            pl.BlockSpec((H1, 16), lambda i: (0, 0)),
        ],
        out_specs=[
            pl.BlockSpec((8, BN, 64), lambda i: (0, i, 0)),
            pl.BlockSpec((BN, 16), lambda i: (i, 0)),
            pl.BlockSpec((BN, 16), lambda i: (i, 0)),
        ],
        out_shape=[
            jax.ShapeDtypeStruct((8, NPAD, 64), F32),
            jax.ShapeDtypeStruct((NPAD, 16), F32),
            jax.ShapeDtypeStruct((NPAD, 16), F32),
        ],
    )(x_pad, W1, afs1, afd1)

    hflat = h1c.reshape(8 * NPAD, 64)

    sce = pl.kernel(
        functools.partial(_sce_body, RW),
        out_type=jax.ShapeDtypeStruct((epad, 16), F32),
        mesh=mesh,
        compiler_params=cp,
        scratch_types=[
            pltpu.VMEM((RW, 128), jnp.int32),
            pltpu.VMEM((RW, 128), jnp.int32),
            pltpu.VMEM((128, 16), F32),
            pltpu.VMEM((128, 16), F32),
            pltpu.VMEM((128, 16), F32),
            pltpu.VMEM((128,), jnp.int32),
            pltpu.VMEM((48,), jnp.int32),
        ],
    )

    def _exp_map(e):
        return pl.pallas_call(
            _tcx_body,
            grid=(epad // BNE,),
            in_specs=[pl.BlockSpec((BNE, 16), lambda i: (i, 0))],
            out_specs=pl.BlockSpec((BNE, 16), lambda i: (i, 0)),
            out_shape=jax.ShapeDtypeStruct((epad, 16), F32),
        )(e)

    e1 = sce(as1, ad1, srcf, dstf)
    ex1 = _exp_map(e1)

    scm1 = pl.kernel(
        functools.partial(_scm1_body, RT),
        out_type=jax.ShapeDtypeStruct((8, NPAD, 64), F32),
        mesh=mesh,
        compiler_params=cp,
        scratch_types=[
            pltpu.VMEM_SHARED((NPAD, 16), F32),
            pltpu.VMEM_SHARED((NPAD, 64), F32),
            pltpu.VMEM((RT, 128), jnp.int32),
            pltpu.VMEM((RT, 128), jnp.int32),
            pltpu.VMEM((128, 16), F32),
            pltpu.VMEM((128, 16), F32),
            pltpu.VMEM((128, 64), F32),
            pltpu.VMEM((128,), jnp.int32),
            pltpu.VMEM((128,), jnp.int32),
            pltpu.VMEM((48,), jnp.int32),
        ],
    )
    out1 = scm1(hflat, ex1, srcf, dstf, z16, z64)

    h1cat = out1.transpose(1, 0, 2).reshape(NPAD, H1)
    b1r = b1.reshape(1, H1)
    w2p = jnp.pad(W2, ((0, 0), (0, C2 - NCLASS)))
    afs2 = jnp.zeros((C2, 16), F32).at[:NCLASS, 0].set(
        att_src2.reshape(NCLASS))
    afd2 = jnp.zeros((C2, 16), F32).at[:NCLASS, 0].set(
        att_dst2.reshape(NCLASS))

    gpad, as2, ad2 = pl.pallas_call(
        _tc2_body,
        grid=(grid,),
        in_specs=[
            pl.BlockSpec((BN, H1), lambda i: (i, 0)),
            pl.BlockSpec((1, H1), lambda i: (0, 0)),
            pl.BlockSpec((H1, C2), lambda i: (0, 0)),
            pl.BlockSpec((C2, 16), lambda i: (0, 0)),
            pl.BlockSpec((C2, 16), lambda i: (0, 0)),
        ],
        out_specs=[
            pl.BlockSpec((BN, C2), lambda i: (i, 0)),
            pl.BlockSpec((BN, 16), lambda i: (i, 0)),
            pl.BlockSpec((BN, 16), lambda i: (i, 0)),
        ],
        out_shape=[
            jax.ShapeDtypeStruct((NPAD, C2), F32),
            jax.ShapeDtypeStruct((NPAD, 16), F32),
            jax.ShapeDtypeStruct((NPAD, 16), F32),
        ],
    )(h1cat, b1r, w2p, afs2, afd2)

    e2 = sce(as2, ad2, srcf, dstf)
    ex2 = _exp_map(e2)

    scm2 = pl.kernel(
        functools.partial(_scm2_body, RT, RW),
        out_type=jax.ShapeDtypeStruct((2, NPAD, C2), F32),
        mesh=mesh,
        compiler_params=cp,
        scratch_types=[
            pltpu.VMEM_SHARED((NPAD, 16), F32),
            pltpu.VMEM_SHARED((NPAD, C2), F32),
            pltpu.VMEM((RT, 128), jnp.int32),
            pltpu.VMEM((RW, 128), jnp.int32),
            pltpu.VMEM((RW, 128), jnp.int32),
            pltpu.VMEM((128, 16), F32),
            pltpu.VMEM((128, 16), F32),
            pltpu.VMEM((128, C2), F32),
            pltpu.VMEM((128,), jnp.int32),
            pltpu.VMEM((48,), jnp.int32),
        ],
    )
    out2p = scm2(gpad, ex2, srcf, dstf, z16, z48)

    b2r = b2.reshape(1, NCLASS)
    out2f = out2p.reshape(2 * NPAD, C2)
    nb = NPAD // BN
    out = pl.pallas_call(
        _tc3_body,
        grid=(grid,),
        in_specs=[
            pl.BlockSpec((BN, C2), lambda i: (i, 0)),
            pl.BlockSpec((BN, C2), lambda i: (i + nb, 0)),
            pl.BlockSpec((1, NCLASS), lambda i: (0, 0)),
        ],
        out_specs=pl.BlockSpec((BN, NCLASS), lambda i: (i, 0)),
        out_shape=jax.ShapeDtypeStruct((NPAD, NCLASS), F32),
    )(out2f, out2f, b2r)

    return out[:n]
